# Initial kernel scaffold; baseline (speedup 1.0000x reference)
import functools

import jax
import jax.numpy as jnp
from jax import lax
from jax.experimental import pallas as pl
from jax.experimental.pallas import tpu as pltpu
from jax.experimental.pallas import tpu_sc as plsc

F32 = jnp.float32
D = 16
IC = 128
STAGE = 1280
NW = 32


@functools.lru_cache(maxsize=None)
def _sc_gather(E_pad: int, N: int):
    per_tile = E_pad // NW
    n_stage = per_tile // STAGE
    inner = STAGE // IC
    mesh = plsc.VectorSubcoreMesh(core_axis_name="c", subcore_axis_name="s")

    @functools.partial(
        pl.kernel,
        out_type=jax.ShapeDtypeStruct((E_pad, D), F32),
        mesh=mesh,
        scratch_types=[
            pltpu.VMEM((inner, IC), jnp.int32),
            pltpu.VMEM((STAGE, D), F32),
            pltpu.SemaphoreType.DMA,
        ],
    )
    def k(table_hbm, idx_hbm, out_hbm, idx_v, rows_v, sem):
        wid = lax.axis_index("c") * 16 + lax.axis_index("s")
        base_irow = wid * (per_tile // IC)

        def stage_body(st, _):
            pltpu.sync_copy(idx_hbm.at[pl.ds(base_irow + st * inner, inner), :],
                            idx_v)
            cps = [
                pltpu.async_copy(
                    table_hbm.at[idx_v.at[j]],
                    rows_v.at[pl.ds(j * IC, IC), :],
                    sem,
                )
                for j in range(inner)
            ]
            for cp in cps:
                cp.wait()
            e0 = wid * per_tile + st * STAGE
            pltpu.sync_copy(rows_v, out_hbm.at[pl.ds(e0, STAGE), :])
            return 0

        lax.fori_loop(0, n_stage, stage_body, 0)

    return k


@functools.lru_cache(maxsize=None)
def _sc_scatter_add(E_pad: int, N: int, N_acc: int):
    per_tile = E_pad // NW
    n_stage = per_tile // STAGE
    inner = STAGE // IC
    rpt = N_acc // 16
    zb = min(rpt, 1280)
    n_zero = rpt // zb
    dpt = N // 16
    db = min(dpt, 1250)
    n_dump = dpt // db
    mesh = plsc.VectorSubcoreMesh(core_axis_name="c", subcore_axis_name="s")

    @functools.partial(
        pl.kernel,
        out_type=jax.ShapeDtypeStruct((2, N, D), F32),
        mesh=mesh,
        scratch_types=[
            pltpu.VMEM((inner, IC), jnp.int32),
            pltpu.VMEM((STAGE, D), F32),
            pltpu.VMEM((zb, D), F32),
            pltpu.VMEM_SHARED((N_acc, D), F32),
            pltpu.SemaphoreType.DMA,
        ],
    )
    def k(msg_hbm, idx_hbm, out_hbm, idx_v, rows_v, zbuf, acc, sem):
        c = lax.axis_index("c")
        s = lax.axis_index("s")
        wid = c * 16 + s

        def zfill(i, _):
            zbuf[i] = jnp.zeros((D,), F32)
            return 0
        lax.fori_loop(0, zb, zfill, 0)

        def zcopy(i, _):
            pltpu.sync_copy(zbuf, acc.at[pl.ds(s * rpt + i * zb, zb), :])
            return 0
        lax.fori_loop(0, n_zero, zcopy, 0)
        plsc.subcore_barrier()

        def stage_body(st, _):
            pltpu.sync_copy(
                idx_hbm.at[pl.ds(wid * (per_tile // IC) + st * inner, inner), :],
                idx_v)
            e0 = wid * per_tile + st * STAGE
            pltpu.sync_copy(msg_hbm.at[pl.ds(e0, STAGE), :], rows_v)
            for j in range(inner):
                pltpu.sync_copy(
                    rows_v.at[pl.ds(j * IC, IC), :],
                    acc.at[idx_v.at[j]],
                    add=True,
                )
            return 0

        lax.fori_loop(0, n_stage, stage_body, 0)
        plsc.subcore_barrier()

        def dump(i, _):
            r0 = s * dpt + i * db
            pltpu.sync_copy(acc.at[pl.ds(r0, db), :], rows_v.at[pl.ds(0, db), :])
            pltpu.sync_copy(rows_v.at[pl.ds(0, db), :],
                            out_hbm.at[c, pl.ds(r0, db), :])
            return 0

        lax.fori_loop(0, n_dump, dump, 0)

    return k


@functools.lru_cache(maxsize=None)
def _sc_count(E_pad: int, N: int, N_acc: int):
    per_tile = E_pad // NW
    n_stage = per_tile // STAGE
    inner = STAGE // IC
    rpt = N_acc // 16
    zb = min(rpt, 1280)
    n_zero = rpt // zb
    dpt = N // 16
    db = min(dpt, 1250)
    n_dump = dpt // db
    mesh = plsc.VectorSubcoreMesh(core_axis_name="c", subcore_axis_name="s")

    @functools.partial(
        pl.kernel,
        out_type=jax.ShapeDtypeStruct((2, N, D), F32),
        mesh=mesh,
        scratch_types=[
            pltpu.VMEM((inner, IC), jnp.int32),
            pltpu.VMEM((IC, D), F32),
            pltpu.VMEM((max(zb, db), D), F32),
            pltpu.VMEM_SHARED((N_acc, D), F32),
            pltpu.SemaphoreType.DMA,
        ],
    )
    def k(idx_hbm, out_hbm, idx_v, ones_v, zbuf, acc, sem):
        c = lax.axis_index("c")
        s = lax.axis_index("s")
        wid = c * 16 + s

        def ofill(i, _):
            ones_v[i] = jnp.ones((D,), F32)
            return 0
        lax.fori_loop(0, IC, ofill, 0)

        def zfill(i, _):
            zbuf[i] = jnp.zeros((D,), F32)
            return 0
        lax.fori_loop(0, zb, zfill, 0)

        def zcopy(i, _):
            pltpu.sync_copy(zbuf.at[pl.ds(0, zb), :],
                            acc.at[pl.ds(s * rpt + i * zb, zb), :])
            return 0
        lax.fori_loop(0, n_zero, zcopy, 0)
        plsc.subcore_barrier()

        def stage_body(st, _):
            pltpu.sync_copy(
                idx_hbm.at[pl.ds(wid * (per_tile // IC) + st * inner, inner), :],
                idx_v)
            for j in range(inner):
                pltpu.sync_copy(ones_v, acc.at[idx_v.at[j]], add=True)
            return 0

        lax.fori_loop(0, n_stage, stage_body, 0)
        plsc.subcore_barrier()

        def dump(i, _):
            r0 = s * dpt + i * db
            pltpu.sync_copy(acc.at[pl.ds(r0, db), :], zbuf.at[pl.ds(0, db), :])
            pltpu.sync_copy(zbuf.at[pl.ds(0, db), :],
                            out_hbm.at[c, pl.ds(r0, db), :])
            return 0

        lax.fori_loop(0, n_dump, dump, 0)

    return k


def _dot(a, b):
    return jnp.dot(a, b, preferred_element_type=F32)


def _prep_body(x_ref, w_ref, b_ref, lw_ref, lb_ref, out0_ref, lin_ref):
    x = x_ref[...]
    out0_ref[...] = jnp.maximum(_dot(x, w_ref[...]) + b_ref[...], 0.0)
    lin_ref[...] = _dot(x, lw_ref[...]) + lb_ref[...]


def _prep(x, W, b, lW, lb):
    N, Fin = x.shape
    BN = 2000
    full = lambda r, c: pl.BlockSpec((r, c), lambda i: (0, 0))
    return pl.pallas_call(
        _prep_body,
        grid=(N // BN,),
        in_specs=[pl.BlockSpec((BN, Fin), lambda i: (i, 0)),
                  full(Fin, D), full(1, D), full(Fin, D), full(1, D)],
        out_specs=[pl.BlockSpec((BN, D), lambda i: (i, 0)),
                   pl.BlockSpec((BN, D), lambda i: (i, 0))],
        out_shape=[jax.ShapeDtypeStruct((N, D), F32),
                   jax.ShapeDtypeStruct((N, D), F32)],
    )(x, W, b, lW, lb)


def _msg_body(ea_ref, xj_ref, w1_ref, b1_ref, w2_ref, b2_ref, r_ref, s_ref,
              msg_ref):
    e1 = jnp.maximum(_dot(ea_ref[...], w1_ref[...]) + b1_ref[...], 0.0)
    e2 = jnp.maximum(_dot(e1, w2_ref[...]) + b2_ref[...], 0.0)
    xb = _dot(xj_ref[...], r_ref[...])
    msg_ref[...] = _dot(xb * e2, s_ref[...])


def _msg(ea, xj, W1, b1, W2, b2, Rm, Sm):
    E, EF = ea.shape
    BE = 2560
    full = lambda r, c: pl.BlockSpec((r, c), lambda i: (0, 0))
    return pl.pallas_call(
        _msg_body,
        grid=(E // BE,),
        in_specs=[pl.BlockSpec((BE, EF), lambda i: (i, 0)),
                  pl.BlockSpec((BE, D), lambda i: (i, 0)),
                  full(EF, D), full(1, D), full(D, D * D), full(1, D * D),
                  full(D, D * D), full(D * D, D)],
        out_specs=pl.BlockSpec((BE, D), lambda i: (i, 0)),
        out_shape=jax.ShapeDtypeStruct((E, D), F32),
    )(ea, xj, W1, b1, W2, b2, Rm, Sm)


def _gru_body(p_ref, cp_ref, h_ref, root_ref, bias_ref,
              wir_ref, wiz_ref, win_ref, whr_ref, whz_ref, whn_ref,
              lin_ref, out_ref, *, add_lin):
    h = h_ref[...]
    cnt = jnp.maximum(cp_ref[0] + cp_ref[1], 1.0)
    agg = (p_ref[0] + p_ref[1]) / cnt
    m = jnp.maximum(agg + _dot(h, root_ref[...]) + bias_ref[...], 0.0)
    r = jax.nn.sigmoid(_dot(m, wir_ref[...]) + _dot(h, whr_ref[...]))
    z = jax.nn.sigmoid(_dot(m, wiz_ref[...]) + _dot(h, whz_ref[...]))
    n = jnp.tanh(_dot(m, win_ref[...]) + r * _dot(h, whn_ref[...]))
    hn = (1.0 - z) * n + z * h
    if add_lin:
        hn = hn + lin_ref[...]
    out_ref[...] = hn


def _gru(parts, cparts, h, root, bias, gih, ghh, lin, add_lin):
    N = h.shape[0]
    BN = 2000
    full = lambda r, c: pl.BlockSpec((r, c), lambda i: (0, 0))
    body = functools.partial(_gru_body, add_lin=add_lin)
    return pl.pallas_call(
        body,
        grid=(N // BN,),
        in_specs=[pl.BlockSpec((2, BN, D), lambda i: (0, i, 0)),
                  pl.BlockSpec((2, BN, D), lambda i: (0, i, 0)),
                  pl.BlockSpec((BN, D), lambda i: (i, 0)),
                  full(D, D), full(1, D),
                  full(D, D), full(D, D), full(D, D),
                  full(D, D), full(D, D), full(D, D),
                  pl.BlockSpec((BN, D), lambda i: (i, 0))],
        out_specs=pl.BlockSpec((BN, D), lambda i: (i, 0)),
        out_shape=jax.ShapeDtypeStruct((N, D), F32),
    )(parts, cparts, h, root, bias, *gih, *ghh, lin)


def _att_body(x_ref, lx0, lx1, lx2, lx3, lx4, lx5, lx6, lx7,
              mw_ref, mb_ref, lnw_ref, lnb_ref,
              wk_ref, bk_ref, wv_ref, bv_ref, wq_ref, bq_ref,
              wf_ref, bf_ref, hsel_ref, hselt_ref, xc_ref):
    lxs = (lx0, lx1, lx2, lx3, lx4, lx5, lx6, lx7)
    x = x_ref[...]
    xq = jnp.maximum(_dot(x, mw_ref[...]) + mb_ref[...], 0.0)
    q = (_dot(xq, wq_ref[...]) + bq_ref[...]) * 0.5
    hsel = hsel_ref[...]
    logits = []
    for j in range(8):
        kj = _dot(lxs[j][...], wk_ref[...]) + bk_ref[...]
        logits.append(_dot(q * kj, hsel))
    m = logits[0]
    for j in range(1, 8):
        m = jnp.maximum(m, logits[j])
    ps = [jnp.exp(l - m) for l in logits]
    den = ps[0]
    for j in range(1, 8):
        den = den + ps[j]
    hselt = hselt_ref[...]
    ctx = jnp.zeros_like(q)
    for j in range(8):
        vj = _dot(lxs[j][...], wv_ref[...]) + bv_ref[...]
        ctx = ctx + _dot(ps[j] / den, hselt) * vj
    low_ctx = _dot(ctx, wf_ref[...]) + bf_ref[...]
    left = jnp.maximum(_dot(x, lnw_ref[...]) + lnb_ref[...], 0.0)
    xc_ref[...] = jnp.concatenate([left, low_ctx], axis=1)


def _attention(x, lxs, mW, mb, lnW, lnb, att, Hsel, HselT):
    N = x.shape[0]
    BN = 2000
    full = lambda r, c: pl.BlockSpec((r, c), lambda i: (0, 0))
    blk = pl.BlockSpec((BN, D), lambda i: (i, 0))
    return pl.pallas_call(
        _att_body,
        grid=(N // BN,),
        in_specs=[blk] + [blk] * 8 +
                 [full(D, D), full(1, D), full(D, D), full(1, D),
                  full(D, D), full(1, D), full(D, D), full(1, D),
                  full(D, D), full(1, D), full(D, D), full(1, D),
                  full(D, 4), full(4, D)],
        out_specs=pl.BlockSpec((BN, 2 * D), lambda i: (i, 0)),
        out_shape=jax.ShapeDtypeStruct((N, 2 * D), F32),
    )(x, *lxs, mW, mb, lnW, lnb,
      att['Wk'], att['bk'][None], att['Wv'], att['bv'][None],
      att['Wq'], att['bq'][None], att['Wf'], att['bf'][None], Hsel, HselT)


def _s2s_body(x_ref,
              wii_ref, wif_ref, wig_ref, wio_ref,
              whi_ref, whf_ref, whg_ref, who_ref,
              bi_ref, bf_ref, bg_ref, bo_ref,
              m1w_ref, m1b_ref, m2w_ref, m2b_ref, out_ref, *, n_events):
    X = x_ref[...]
    N = X.shape[0]
    seg = N // n_events
    rI = lax.broadcasted_iota(jnp.int32, (n_events, N), 0)
    cI = lax.broadcasted_iota(jnp.int32, (n_events, N), 1)
    mb = rI == (cI // seg)
    q_star = jnp.zeros((n_events, 2 * D), F32)
    h = jnp.zeros((n_events, D), F32)
    c = jnp.zeros((n_events, D), F32)
    for _ in range(3):
        gi = _dot(q_star, wii_ref[...]) + _dot(h, whi_ref[...]) + bi_ref[...]
        gf = _dot(q_star, wif_ref[...]) + _dot(h, whf_ref[...]) + bf_ref[...]
        gg = _dot(q_star, wig_ref[...]) + _dot(h, whg_ref[...]) + bg_ref[...]
        go = _dot(q_star, wio_ref[...]) + _dot(h, who_ref[...]) + bo_ref[...]
        c = jax.nn.sigmoid(gf) * c + jax.nn.sigmoid(gi) * jnp.tanh(gg)
        h = jax.nn.sigmoid(go) * jnp.tanh(c)
        eT = lax.dot_general(h, X, (((1,), (1,)), ((), ())),
                             preferred_element_type=F32)
        e_m = jnp.where(mb, eT, -1e30)
        emax = jnp.max(e_m, axis=1, keepdims=True)
        p = jnp.where(mb, jnp.exp(eT - emax), 0.0)
        den = jnp.sum(p, axis=1, keepdims=True)
        r = _dot(p / den, X)
        q_star = jnp.concatenate([h, r], axis=1)
    o = jnp.maximum(_dot(q_star, m1w_ref[...]) + m1b_ref[...], 0.0)
    o2 = _dot(o, m2w_ref[...]) + m2b_ref[...]
    mx = jnp.max(o2, axis=1, keepdims=True)
    lse = jnp.log(jnp.sum(jnp.exp(o2 - mx), axis=1, keepdims=True)) + mx
    out_ref[...] = o2 - lse


def _set2set(h2, s2s, m1W, m1b, m2W, m2b, n_events):
    N = h2.shape[0]
    body = functools.partial(_s2s_body, n_events=n_events)
    full = lambda r, c: pl.BlockSpec((r, c), lambda i: (0, 0))
    WihT = s2s['Wih'].T
    WhhT = s2s['Whh'].T
    bsum = (s2s['bih'] + s2s['bhh'])[None]
    gates = [(WihT[:, g * D:(g + 1) * D], WhhT[:, g * D:(g + 1) * D],
              bsum[:, g * D:(g + 1) * D]) for g in range(4)]
    (wii, whi, bi), (wif, whf, bf), (wig, whg, bg), (wio, who, bo) = gates
    return pl.pallas_call(
        body,
        grid=(1,),
        in_specs=[full(N, D)] +
                 [full(2 * D, D)] * 4 + [full(D, D)] * 4 + [full(1, D)] * 4 +
                 [full(2 * D, D), full(1, D), full(D, 4), full(1, 4)],
        out_specs=full(n_events, 4),
        out_shape=jax.ShapeDtypeStruct((n_events, 4), F32),
    )(h2, wii, wif, wig, wio, whi, whf, whg, who, bi, bf, bg, bo,
      m1W, m1b[None], m2W, m2b[None])


def _fold_bn(W, b, g, bb):
    s = (g / jnp.sqrt(jnp.float32(1.0 + 1e-5))).astype(F32)
    return W * s[None, :], (b * s + bb)[None, :]


def _pad_edges(edge_index, edge_attr, E_pad, N):
    E = edge_index.shape[1]
    pad = E_pad - E
    src = jnp.concatenate([edge_index[0], jnp.zeros((pad,), jnp.int32)])
    dst = jnp.concatenate([edge_index[1], jnp.full((pad,), N, jnp.int32)])
    ea = jnp.concatenate([edge_attr, jnp.zeros((pad, edge_attr.shape[1]), F32)])
    return src.reshape(E_pad // IC, IC), dst.reshape(E_pad // IC, IC), ea


def _conv(x_nodes, src2d, dst2d, ea_pad, p, N, E_pad, N_acc):
    W1, b1 = _fold_bn(p['en1_W'], p['en1_b'], p['en_bn_g'], p['en_bn_b'])
    Wm, bm = _fold_bn(p['mlp_W'], p['mlp_b'], p['bn_g'], p['bn_b'])
    out0, lin = _prep(x_nodes, Wm, bm, p['lin_W'], p['lin_b'][None])
    cparts = _sc_count(E_pad, N, N_acc)(dst2d)
    di = jnp.arange(D * D, dtype=jnp.int32)
    Rm = (jnp.arange(D)[:, None] == (di[None, :] // D)).astype(F32)
    Sm = ((di[:, None] % D) == jnp.arange(D)[None, :]).astype(F32)
    gih = (p['gru_Wih'][0:D].T, p['gru_Wih'][D:2 * D].T, p['gru_Wih'][2 * D:].T)
    ghh = (p['gru_Whh'][0:D].T, p['gru_Whh'][D:2 * D].T, p['gru_Whh'][2 * D:].T)
    h = out0
    for step in range(3):
        xj = _sc_gather(E_pad, N)(h, src2d)
        msg = _msg(ea_pad, xj, W1, b1, p['en2_W'], p['en2_b'][None], Rm, Sm)
        parts = _sc_scatter_add(E_pad, N, N_acc)(msg, dst2d)
        h = _gru(parts, cparts, h, p['root'], p['bias'][None], gih, ghh,
                 lin, add_lin=(step == 2))
    return h


def kernel(x, edge_index, edge_attr, batch, low_x, low_edge_index,
           low_edge_attr, low_batch, params):
    p = params
    N_LOW = low_x.shape[0]
    N_HIGH = x.shape[0]
    n_events = 100
    EPL = 327680
    EPH = 163840
    NAL = 81920
    NAH = 11264

    lsrc, ldst, lea = _pad_edges(low_edge_index, low_edge_attr, EPL, N_LOW)
    hsrc, hdst, hea = _pad_edges(edge_index, edge_attr, EPH, N_HIGH)

    lx = _conv(low_x, lsrc, ldst, lea, p['c1'], N_LOW, EPL, NAL)
    lx3 = lx.reshape(N_HIGH, 8, D)
    lxs = [lx3[:, j, :] for j in range(8)]

    Hsel = ((jnp.arange(D)[:, None] // 4) == jnp.arange(4)[None, :]).astype(F32)
    xc = _attention(x, lxs, p['mlp_W'], p['mlp_b'][None],
                    p['ln_W'], p['ln_b'][None], p['att'], Hsel, Hsel.T)

    h2 = _conv(xc, hsrc, hdst, hea, p['c2'], N_HIGH, EPH, NAH)

    return _set2set(h2, p['s2s'], p['mlp1_W'], p['mlp1_b'],
                    p['mlp2_W'], p['mlp2_b'], n_events)


# speedup vs baseline: 2.7885x; 2.7885x over previous
import functools

import jax
import jax.numpy as jnp
from jax import lax
from jax.experimental import pallas as pl
from jax.experimental.pallas import tpu as pltpu
from jax.experimental.pallas import tpu_sc as plsc

F32 = jnp.float32
D = 16
IC = 128
STAGE = 1024
NW = 32


@functools.lru_cache(maxsize=None)
def _sc_gather(E_pad: int, N: int):
    per_tile = E_pad // NW
    n_stage = per_tile // STAGE
    inner = STAGE // IC
    mesh = plsc.VectorSubcoreMesh(core_axis_name="c", subcore_axis_name="s")

    @functools.partial(
        pl.kernel,
        out_type=jax.ShapeDtypeStruct((E_pad, D), F32),
        mesh=mesh,
        compiler_params=pltpu.CompilerParams(use_tc_tiling_on_sc=False),
        scratch_types=[
            pltpu.VMEM((inner, IC), jnp.int32),
            pltpu.VMEM((STAGE, D), F32),
            pltpu.SemaphoreType.DMA,
        ],
    )
    def k(table_hbm, idx_hbm, out_hbm, idx_v, rows_v, sem):
        wid = lax.axis_index("c") * 16 + lax.axis_index("s")
        base_irow = wid * (per_tile // IC)

        def stage_body(st, _):
            pltpu.sync_copy(idx_hbm.at[pl.ds(base_irow + st * inner, inner), :],
                            idx_v)
            cps = [
                pltpu.async_copy(
                    table_hbm.at[idx_v.at[j]],
                    rows_v.at[pl.ds(j * IC, IC), :],
                    sem,
                )
                for j in range(inner)
            ]
            for cp in cps:
                cp.wait()
            e0 = wid * per_tile + st * STAGE
            pltpu.sync_copy(rows_v, out_hbm.at[pl.ds(e0, STAGE), :])
            return 0

        lax.fori_loop(0, n_stage, stage_body, 0)

    return k


@functools.lru_cache(maxsize=None)
def _sc_scatter_add(E_pad: int, N: int, N_acc: int):
    per_tile = E_pad // NW
    n_stage = per_tile // STAGE
    inner = STAGE // IC
    rpt = N_acc // 16
    zb = min(rpt, STAGE)
    n_zero = rpt // zb
    mesh = plsc.VectorSubcoreMesh(core_axis_name="c", subcore_axis_name="s")

    @functools.partial(
        pl.kernel,
        out_type=jax.ShapeDtypeStruct((2, N_acc, D), F32),
        mesh=mesh,
        compiler_params=pltpu.CompilerParams(use_tc_tiling_on_sc=False),
        scratch_types=[
            pltpu.VMEM((inner, IC), jnp.int32),
            pltpu.VMEM((STAGE, D), F32),
            pltpu.VMEM((zb, D), F32),
            pltpu.VMEM_SHARED((N_acc, D), F32),
            pltpu.SemaphoreType.DMA,
        ],
    )
    def k(msg_hbm, idx_hbm, out_hbm, idx_v, rows_v, zbuf, acc, sem):
        c = lax.axis_index("c")
        s = lax.axis_index("s")
        wid = c * 16 + s

        def zfill(i, _):
            zbuf[i] = jnp.zeros((D,), F32)
            return 0
        lax.fori_loop(0, zb, zfill, 0)

        def zcopy(i, _):
            pltpu.sync_copy(zbuf, acc.at[pl.ds(s * rpt + i * zb, zb), :])
            return 0
        lax.fori_loop(0, n_zero, zcopy, 0)
        plsc.subcore_barrier()

        def stage_body(st, _):
            pltpu.sync_copy(
                idx_hbm.at[pl.ds(wid * (per_tile // IC) + st * inner, inner), :],
                idx_v)
            e0 = wid * per_tile + st * STAGE
            pltpu.sync_copy(msg_hbm.at[pl.ds(e0, STAGE), :], rows_v)
            for j in range(inner):
                pltpu.sync_copy(
                    rows_v.at[pl.ds(j * IC, IC), :],
                    acc.at[idx_v.at[j]],
                    add=True,
                )
            return 0

        lax.fori_loop(0, n_stage, stage_body, 0)
        plsc.subcore_barrier()

        def dump(i, _):
            r0 = s * rpt + i * zb
            pltpu.sync_copy(acc.at[pl.ds(r0, zb), :], rows_v.at[pl.ds(0, zb), :])
            pltpu.sync_copy(rows_v.at[pl.ds(0, zb), :],
                            out_hbm.at[c, pl.ds(r0, zb), :])
            return 0

        lax.fori_loop(0, n_zero, dump, 0)

    return k


@functools.lru_cache(maxsize=None)
def _sc_count(E_pad: int, N: int, N_acc: int):
    per_tile = E_pad // NW
    n_stage = per_tile // STAGE
    inner = STAGE // IC
    rpt = N_acc // 16
    zb = min(rpt, STAGE)
    n_zero = rpt // zb
    mesh = plsc.VectorSubcoreMesh(core_axis_name="c", subcore_axis_name="s")

    @functools.partial(
        pl.kernel,
        out_type=jax.ShapeDtypeStruct((2, N_acc, D), F32),
        mesh=mesh,
        compiler_params=pltpu.CompilerParams(use_tc_tiling_on_sc=False),
        scratch_types=[
            pltpu.VMEM((inner, IC), jnp.int32),
            pltpu.VMEM((IC, D), F32),
            pltpu.VMEM((zb, D), F32),
            pltpu.VMEM_SHARED((N_acc, D), F32),
            pltpu.SemaphoreType.DMA,
        ],
    )
    def k(idx_hbm, out_hbm, idx_v, ones_v, zbuf, acc, sem):
        c = lax.axis_index("c")
        s = lax.axis_index("s")
        wid = c * 16 + s

        def ofill(i, _):
            ones_v[i] = jnp.ones((D,), F32)
            return 0
        lax.fori_loop(0, IC, ofill, 0)

        def zfill(i, _):
            zbuf[i] = jnp.zeros((D,), F32)
            return 0
        lax.fori_loop(0, zb, zfill, 0)

        def zcopy(i, _):
            pltpu.sync_copy(zbuf.at[pl.ds(0, zb), :],
                            acc.at[pl.ds(s * rpt + i * zb, zb), :])
            return 0
        lax.fori_loop(0, n_zero, zcopy, 0)
        plsc.subcore_barrier()

        def stage_body(st, _):
            pltpu.sync_copy(
                idx_hbm.at[pl.ds(wid * (per_tile // IC) + st * inner, inner), :],
                idx_v)
            for j in range(inner):
                pltpu.sync_copy(ones_v, acc.at[idx_v.at[j]], add=True)
            return 0

        lax.fori_loop(0, n_stage, stage_body, 0)
        plsc.subcore_barrier()

        def dump(i, _):
            r0 = s * rpt + i * zb
            pltpu.sync_copy(acc.at[pl.ds(r0, zb), :], zbuf)
            pltpu.sync_copy(zbuf, out_hbm.at[c, pl.ds(r0, zb), :])
            return 0

        lax.fori_loop(0, n_zero, dump, 0)

    return k


def _dot(a, b):
    return jnp.dot(a, b, preferred_element_type=F32)


def _prep_body(x_ref, w_ref, b_ref, lw_ref, lb_ref, out0_ref, lin_ref):
    x = x_ref[...]
    out0_ref[...] = jnp.maximum(_dot(x, w_ref[...]) + b_ref[...], 0.0)
    lin_ref[...] = _dot(x, lw_ref[...]) + lb_ref[...]


def _prep(x, W, b, lW, lb):
    N, Fin = x.shape
    BN = 2000
    full = lambda r, c: pl.BlockSpec((r, c), lambda i: (0, 0))
    return pl.pallas_call(
        _prep_body,
        grid=(N // BN,),
        in_specs=[pl.BlockSpec((BN, Fin), lambda i: (i, 0)),
                  full(Fin, D), full(1, D), full(Fin, D), full(1, D)],
        out_specs=[pl.BlockSpec((BN, D), lambda i: (i, 0)),
                   pl.BlockSpec((BN, D), lambda i: (i, 0))],
        out_shape=[jax.ShapeDtypeStruct((N, D), F32),
                   jax.ShapeDtypeStruct((N, D), F32)],
    )(x, W, b, lW, lb)


def _msg_body(ea_ref, xj_ref, w1_ref, b1_ref, w2_ref, b2_ref, r_ref, s_ref,
              msg_ref):
    e1 = jnp.maximum(_dot(ea_ref[...], w1_ref[...]) + b1_ref[...], 0.0)
    e2 = jnp.maximum(_dot(e1, w2_ref[...]) + b2_ref[...], 0.0)
    xb = _dot(xj_ref[...], r_ref[...])
    msg_ref[...] = _dot(xb * e2, s_ref[...])


def _msg(ea, xj, W1, b1, W2, b2, Rm, Sm):
    E, EF = ea.shape
    BE = 2560
    full = lambda r, c: pl.BlockSpec((r, c), lambda i: (0, 0))
    return pl.pallas_call(
        _msg_body,
        grid=(E // BE,),
        in_specs=[pl.BlockSpec((BE, EF), lambda i: (i, 0)),
                  pl.BlockSpec((BE, D), lambda i: (i, 0)),
                  full(EF, D), full(1, D), full(D, D * D), full(1, D * D),
                  full(D, D * D), full(D * D, D)],
        out_specs=pl.BlockSpec((BE, D), lambda i: (i, 0)),
        out_shape=jax.ShapeDtypeStruct((E, D), F32),
    )(ea, xj, W1, b1, W2, b2, Rm, Sm)


def _gru_body(p_ref, cp_ref, h_ref, root_ref, bias_ref,
              wir_ref, wiz_ref, win_ref, whr_ref, whz_ref, whn_ref,
              lin_ref, out_ref, *, add_lin):
    h = h_ref[...]
    cnt = jnp.maximum(cp_ref[0] + cp_ref[1], 1.0)
    agg = (p_ref[0] + p_ref[1]) / cnt
    m = jnp.maximum(agg + _dot(h, root_ref[...]) + bias_ref[...], 0.0)
    r = jax.nn.sigmoid(_dot(m, wir_ref[...]) + _dot(h, whr_ref[...]))
    z = jax.nn.sigmoid(_dot(m, wiz_ref[...]) + _dot(h, whz_ref[...]))
    n = jnp.tanh(_dot(m, win_ref[...]) + r * _dot(h, whn_ref[...]))
    hn = (1.0 - z) * n + z * h
    if add_lin:
        hn = hn + lin_ref[...]
    out_ref[...] = hn


def _gru(parts, cparts, h, root, bias, gih, ghh, lin, add_lin):
    N = h.shape[0]
    BN = 2000
    full = lambda r, c: pl.BlockSpec((r, c), lambda i: (0, 0))
    body = functools.partial(_gru_body, add_lin=add_lin)
    return pl.pallas_call(
        body,
        grid=(N // BN,),
        in_specs=[pl.BlockSpec((2, BN, D), lambda i: (0, i, 0)),
                  pl.BlockSpec((2, BN, D), lambda i: (0, i, 0)),
                  pl.BlockSpec((BN, D), lambda i: (i, 0)),
                  full(D, D), full(1, D),
                  full(D, D), full(D, D), full(D, D),
                  full(D, D), full(D, D), full(D, D),
                  pl.BlockSpec((BN, D), lambda i: (i, 0))],
        out_specs=pl.BlockSpec((BN, D), lambda i: (i, 0)),
        out_shape=jax.ShapeDtypeStruct((N, D), F32),
    )(parts, cparts, h, root, bias, *gih, *ghh, lin)


def _att_body(x_ref, lx0, lx1, lx2, lx3, lx4, lx5, lx6, lx7,
              mw_ref, mb_ref, lnw_ref, lnb_ref,
              wk_ref, bk_ref, wv_ref, bv_ref, wq_ref, bq_ref,
              wf_ref, bf_ref, hsel_ref, hselt_ref, xc_ref):
    lxs = (lx0, lx1, lx2, lx3, lx4, lx5, lx6, lx7)
    x = x_ref[...]
    xq = jnp.maximum(_dot(x, mw_ref[...]) + mb_ref[...], 0.0)
    q = (_dot(xq, wq_ref[...]) + bq_ref[...]) * 0.5
    hsel = hsel_ref[...]
    logits = []
    for j in range(8):
        kj = _dot(lxs[j][...], wk_ref[...]) + bk_ref[...]
        logits.append(_dot(q * kj, hsel))
    m = logits[0]
    for j in range(1, 8):
        m = jnp.maximum(m, logits[j])
    ps = [jnp.exp(l - m) for l in logits]
    den = ps[0]
    for j in range(1, 8):
        den = den + ps[j]
    hselt = hselt_ref[...]
    ctx = jnp.zeros_like(q)
    for j in range(8):
        vj = _dot(lxs[j][...], wv_ref[...]) + bv_ref[...]
        ctx = ctx + _dot(ps[j] / den, hselt) * vj
    low_ctx = _dot(ctx, wf_ref[...]) + bf_ref[...]
    left = jnp.maximum(_dot(x, lnw_ref[...]) + lnb_ref[...], 0.0)
    xc_ref[...] = jnp.concatenate([left, low_ctx], axis=1)


def _attention(x, lxs, mW, mb, lnW, lnb, att, Hsel, HselT):
    N = x.shape[0]
    BN = 2000
    full = lambda r, c: pl.BlockSpec((r, c), lambda i: (0, 0))
    blk = pl.BlockSpec((BN, D), lambda i: (i, 0))
    return pl.pallas_call(
        _att_body,
        grid=(N // BN,),
        in_specs=[blk] + [blk] * 8 +
                 [full(D, D), full(1, D), full(D, D), full(1, D),
                  full(D, D), full(1, D), full(D, D), full(1, D),
                  full(D, D), full(1, D), full(D, D), full(1, D),
                  full(D, 4), full(4, D)],
        out_specs=pl.BlockSpec((BN, 2 * D), lambda i: (i, 0)),
        out_shape=jax.ShapeDtypeStruct((N, 2 * D), F32),
    )(x, *lxs, mW, mb, lnW, lnb,
      att['Wk'], att['bk'][None], att['Wv'], att['bv'][None],
      att['Wq'], att['bq'][None], att['Wf'], att['bf'][None], Hsel, HselT)


def _s2s_body(x_ref,
              wii_ref, wif_ref, wig_ref, wio_ref,
              whi_ref, whf_ref, whg_ref, who_ref,
              bi_ref, bf_ref, bg_ref, bo_ref,
              m1w_ref, m1b_ref, m2w_ref, m2b_ref, out_ref, *, n_events):
    X = x_ref[...]
    N = X.shape[0]
    seg = N // n_events
    rI = lax.broadcasted_iota(jnp.int32, (n_events, N), 0)
    cI = lax.broadcasted_iota(jnp.int32, (n_events, N), 1)
    mb = rI == (cI // seg)
    q_star = jnp.zeros((n_events, 2 * D), F32)
    h = jnp.zeros((n_events, D), F32)
    c = jnp.zeros((n_events, D), F32)
    for _ in range(3):
        gi = _dot(q_star, wii_ref[...]) + _dot(h, whi_ref[...]) + bi_ref[...]
        gf = _dot(q_star, wif_ref[...]) + _dot(h, whf_ref[...]) + bf_ref[...]
        gg = _dot(q_star, wig_ref[...]) + _dot(h, whg_ref[...]) + bg_ref[...]
        go = _dot(q_star, wio_ref[...]) + _dot(h, who_ref[...]) + bo_ref[...]
        c = jax.nn.sigmoid(gf) * c + jax.nn.sigmoid(gi) * jnp.tanh(gg)
        h = jax.nn.sigmoid(go) * jnp.tanh(c)
        eT = lax.dot_general(h, X, (((1,), (1,)), ((), ())),
                             preferred_element_type=F32)
        e_m = jnp.where(mb, eT, -1e30)
        emax = jnp.max(e_m, axis=1, keepdims=True)
        p = jnp.where(mb, jnp.exp(eT - emax), 0.0)
        den = jnp.sum(p, axis=1, keepdims=True)
        r = _dot(p / den, X)
        q_star = jnp.concatenate([h, r], axis=1)
    o = jnp.maximum(_dot(q_star, m1w_ref[...]) + m1b_ref[...], 0.0)
    o2 = _dot(o, m2w_ref[...]) + m2b_ref[...]
    mx = jnp.max(o2, axis=1, keepdims=True)
    lse = jnp.log(jnp.sum(jnp.exp(o2 - mx), axis=1, keepdims=True)) + mx
    out_ref[...] = o2 - lse


def _set2set(h2, s2s, m1W, m1b, m2W, m2b, n_events):
    N = h2.shape[0]
    body = functools.partial(_s2s_body, n_events=n_events)
    full = lambda r, c: pl.BlockSpec((r, c), lambda i: (0, 0))
    WihT = s2s['Wih'].T
    WhhT = s2s['Whh'].T
    bsum = (s2s['bih'] + s2s['bhh'])[None]
    gates = [(WihT[:, g * D:(g + 1) * D], WhhT[:, g * D:(g + 1) * D],
              bsum[:, g * D:(g + 1) * D]) for g in range(4)]
    (wii, whi, bi), (wif, whf, bf), (wig, whg, bg), (wio, who, bo) = gates
    return pl.pallas_call(
        body,
        grid=(1,),
        in_specs=[full(N, D)] +
                 [full(2 * D, D)] * 4 + [full(D, D)] * 4 + [full(1, D)] * 4 +
                 [full(2 * D, D), full(1, D), full(D, 4), full(1, 4)],
        out_specs=full(n_events, 4),
        out_shape=jax.ShapeDtypeStruct((n_events, 4), F32),
    )(h2, wii, wif, wig, wio, whi, whf, whg, who, bi, bf, bg, bo,
      m1W, m1b[None], m2W, m2b[None])


def _fold_bn(W, b, g, bb):
    s = (g / jnp.sqrt(jnp.float32(1.0 + 1e-5))).astype(F32)
    return W * s[None, :], (b * s + bb)[None, :]


def _pad_edges(edge_index, edge_attr, E_pad, N):
    E = edge_index.shape[1]
    pad = E_pad - E
    src = jnp.concatenate([edge_index[0], jnp.zeros((pad,), jnp.int32)])
    dst = jnp.concatenate([edge_index[1], jnp.full((pad,), N, jnp.int32)])
    ea = jnp.concatenate([edge_attr, jnp.zeros((pad, edge_attr.shape[1]), F32)])
    return src.reshape(E_pad // IC, IC), dst.reshape(E_pad // IC, IC), ea


def _conv(x_nodes, src2d, dst2d, ea_pad, p, N, E_pad, N_acc):
    W1, b1 = _fold_bn(p['en1_W'], p['en1_b'], p['en_bn_g'], p['en_bn_b'])
    Wm, bm = _fold_bn(p['mlp_W'], p['mlp_b'], p['bn_g'], p['bn_b'])
    out0, lin = _prep(x_nodes, Wm, bm, p['lin_W'], p['lin_b'][None])
    cparts = _sc_count(E_pad, N, N_acc)(dst2d)
    di = jnp.arange(D * D, dtype=jnp.int32)
    Rm = (jnp.arange(D)[:, None] == (di[None, :] // D)).astype(F32)
    Sm = ((di[:, None] % D) == jnp.arange(D)[None, :]).astype(F32)
    gih = (p['gru_Wih'][0:D].T, p['gru_Wih'][D:2 * D].T, p['gru_Wih'][2 * D:].T)
    ghh = (p['gru_Whh'][0:D].T, p['gru_Whh'][D:2 * D].T, p['gru_Whh'][2 * D:].T)
    h = out0
    for step in range(3):
        xj = _sc_gather(E_pad, N)(h, src2d)
        msg = _msg(ea_pad, xj, W1, b1, p['en2_W'], p['en2_b'][None], Rm, Sm)
        parts = _sc_scatter_add(E_pad, N, N_acc)(msg, dst2d)
        h = _gru(parts, cparts, h, p['root'], p['bias'][None], gih, ghh,
                 lin, add_lin=(step == 2))
    return h


def kernel(x, edge_index, edge_attr, batch, low_x, low_edge_index,
           low_edge_attr, low_batch, params):
    p = params
    N_LOW = low_x.shape[0]
    N_HIGH = x.shape[0]
    n_events = 100
    EPL = 327680
    EPH = 163840
    NAL = 81920
    NAH = 11264

    lsrc, ldst, lea = _pad_edges(low_edge_index, low_edge_attr, EPL, N_LOW)
    hsrc, hdst, hea = _pad_edges(edge_index, edge_attr, EPH, N_HIGH)

    lx = _conv(low_x, lsrc, ldst, lea, p['c1'], N_LOW, EPL, NAL)
    lx3 = lx.reshape(N_HIGH, 8, D)
    lxs = [lx3[:, j, :] for j in range(8)]

    Hsel = ((jnp.arange(D)[:, None] // 4) == jnp.arange(4)[None, :]).astype(F32)
    xc = _attention(x, lxs, p['mlp_W'], p['mlp_b'][None],
                    p['ln_W'], p['ln_b'][None], p['att'], Hsel, Hsel.T)

    h2 = _conv(xc, hsrc, hdst, hea, p['c2'], N_HIGH, EPH, NAH)

    return _set2set(h2, p['s2s'], p['mlp1_W'], p['mlp1_b'],
                    p['mlp2_W'], p['mlp2_b'], n_events)


# speedup vs baseline: 3.0262x; 1.0852x over previous
import functools

import jax
import jax.numpy as jnp
from jax import lax
from jax.experimental import pallas as pl
from jax.experimental.pallas import tpu as pltpu
from jax.experimental.pallas import tpu_sc as plsc

F32 = jnp.float32
D = 16
IC = 128
STAGE = 1024
NW = 32


@functools.lru_cache(maxsize=None)
def _sc_gather(E_pad: int, N: int):
    per_tile = E_pad // NW
    n_stage = per_tile // STAGE
    inner = STAGE // IC
    mesh = plsc.VectorSubcoreMesh(core_axis_name="c", subcore_axis_name="s")

    @functools.partial(
        pl.kernel,
        out_type=jax.ShapeDtypeStruct((E_pad, D), F32),
        mesh=mesh,
        compiler_params=pltpu.CompilerParams(use_tc_tiling_on_sc=False),
        scratch_types=[
            pltpu.VMEM((2, inner, IC), jnp.int32),
            pltpu.VMEM((2, STAGE, D), F32),
            pltpu.SemaphoreType.DMA,
            pltpu.SemaphoreType.DMA,
            pltpu.SemaphoreType.DMA,
        ],
    )
    def k(table_hbm, idx_hbm, out_hbm, idx_v, rows_v, isem, gsem, osem):
        wid = lax.axis_index("c") * 16 + lax.axis_index("s")
        base_irow = wid * (per_tile // IC)

        cpi = {0: pltpu.async_copy(idx_hbm.at[pl.ds(base_irow, inner), :],
                                   idx_v.at[0], isem)}
        cpo = {}
        for st in range(n_stage):
            b = st % 2
            if st + 1 < n_stage:
                cpi[st + 1] = pltpu.async_copy(
                    idx_hbm.at[pl.ds(base_irow + (st + 1) * inner, inner), :],
                    idx_v.at[1 - b], isem)
            cpi[st].wait()
            if st >= 2:
                cpo[st - 2].wait()
            gs = [
                pltpu.async_copy(
                    table_hbm.at[idx_v.at[b, j]],
                    rows_v.at[b, pl.ds(j * IC, IC), :],
                    gsem,
                )
                for j in range(inner)
            ]
            for g in gs:
                g.wait()
            cpo[st] = pltpu.async_copy(
                rows_v.at[b],
                out_hbm.at[pl.ds(wid * per_tile + st * STAGE, STAGE), :],
                osem)
        for st in range(max(0, n_stage - 2), n_stage):
            cpo[st].wait()

    return k


@functools.lru_cache(maxsize=None)
def _sc_scatter_add(E_pad: int, N: int, N_acc: int):
    per_tile = E_pad // NW
    n_stage = per_tile // STAGE
    inner = STAGE // IC
    rpt = N_acc // 16
    zb = 256 if rpt % 256 == 0 else rpt // 4
    n_zero = rpt // zb
    dpt = N_acc // 16
    db = min(dpt, STAGE)
    n_dump = dpt // db
    mesh = plsc.VectorSubcoreMesh(core_axis_name="c", subcore_axis_name="s")

    @functools.partial(
        pl.kernel,
        out_type=jax.ShapeDtypeStruct((2, N_acc, D), F32),
        mesh=mesh,
        compiler_params=pltpu.CompilerParams(use_tc_tiling_on_sc=False),
        scratch_types=[
            pltpu.VMEM((2, inner, IC), jnp.int32),
            pltpu.VMEM((2, STAGE, D), F32),
            pltpu.VMEM((zb, D), F32),
            pltpu.VMEM_SHARED((N_acc, D), F32),
            pltpu.SemaphoreType.DMA,
            pltpu.SemaphoreType.DMA,
            pltpu.SemaphoreType.DMA,
        ],
    )
    def k(msg_hbm, idx_hbm, out_hbm, idx_v, rows_v, zbuf, acc, isem, asem, osem):
        c = lax.axis_index("c")
        s = lax.axis_index("s")
        wid = c * 16 + s

        def zfill(i, _):
            zbuf[i] = jnp.zeros((D,), F32)
            return 0
        lax.fori_loop(0, zb, zfill, 0)
        zcs = [pltpu.async_copy(zbuf, acc.at[pl.ds(s * rpt + i * zb, zb), :],
                                osem)
               for i in range(n_zero)]
        for z in zcs:
            z.wait()
        plsc.subcore_barrier()

        base_irow = wid * (per_tile // IC)

        def load(st, b):
            ci = pltpu.async_copy(
                idx_hbm.at[pl.ds(base_irow + st * inner, inner), :],
                idx_v.at[b], isem)
            cm = pltpu.async_copy(
                msg_hbm.at[pl.ds(wid * per_tile + st * STAGE, STAGE), :],
                rows_v.at[b], osem)
            return ci, cm

        pend = {0: load(0, 0)}
        adds = {}
        for st in range(n_stage):
            b = st % 2
            ci, cm = pend[st]
            ci.wait()
            cm.wait()
            adds[st] = [
                pltpu.async_copy(
                    rows_v.at[b, pl.ds(j * IC, IC), :],
                    acc.at[idx_v.at[b, j]],
                    asem, add=True)
                for j in range(inner)
            ]
            if st >= 1:
                for a in adds[st - 1]:
                    a.wait()
            if st + 1 < n_stage:
                pend[st + 1] = load(st + 1, 1 - b)
        for a in adds[n_stage - 1]:
            a.wait()
        plsc.subcore_barrier()

        cpo = {}
        for i in range(n_dump):
            b = i % 2
            r0 = s * dpt + i * db
            if i >= 2:
                cpo[i - 2].wait()
            pltpu.sync_copy(acc.at[pl.ds(r0, db), :], rows_v.at[b, pl.ds(0, db), :])
            cpo[i] = pltpu.async_copy(rows_v.at[b, pl.ds(0, db), :],
                                      out_hbm.at[c, pl.ds(r0, db), :], osem)
        for i in range(max(0, n_dump - 2), n_dump):
            cpo[i].wait()

    return k


@functools.lru_cache(maxsize=None)
def _sc_count(E_pad: int, N: int, N_acc: int):
    per_tile = E_pad // NW
    n_stage = per_tile // STAGE
    inner = STAGE // IC
    rpt = N_acc // 16
    zb = 256 if rpt % 256 == 0 else rpt // 4
    n_zero = rpt // zb
    mesh = plsc.VectorSubcoreMesh(core_axis_name="c", subcore_axis_name="s")

    @functools.partial(
        pl.kernel,
        out_type=jax.ShapeDtypeStruct((2, N_acc, D), F32),
        mesh=mesh,
        compiler_params=pltpu.CompilerParams(use_tc_tiling_on_sc=False),
        scratch_types=[
            pltpu.VMEM((2, inner, IC), jnp.int32),
            pltpu.VMEM((IC, D), F32),
            pltpu.VMEM((zb, D), F32),
            pltpu.VMEM_SHARED((N_acc, D), F32),
            pltpu.SemaphoreType.DMA,
            pltpu.SemaphoreType.DMA,
            pltpu.SemaphoreType.DMA,
        ],
    )
    def k(idx_hbm, out_hbm, idx_v, ones_v, zbuf, acc, isem, asem, osem):
        c = lax.axis_index("c")
        s = lax.axis_index("s")
        wid = c * 16 + s

        def ofill(i, _):
            ones_v[i] = jnp.ones((D,), F32)
            return 0
        lax.fori_loop(0, IC, ofill, 0)

        def zfill(i, _):
            zbuf[i] = jnp.zeros((D,), F32)
            return 0
        lax.fori_loop(0, zb, zfill, 0)

        zcs = [pltpu.async_copy(zbuf, acc.at[pl.ds(s * rpt + i * zb, zb), :],
                                osem)
               for i in range(n_zero)]
        for z in zcs:
            z.wait()
        plsc.subcore_barrier()

        base_irow = wid * (per_tile // IC)
        cpi = {0: pltpu.async_copy(idx_hbm.at[pl.ds(base_irow, inner), :],
                                   idx_v.at[0], isem)}
        adds = {}
        for st in range(n_stage):
            b = st % 2
            if st + 1 < n_stage:
                cpi[st + 1] = pltpu.async_copy(
                    idx_hbm.at[pl.ds(base_irow + (st + 1) * inner, inner), :],
                    idx_v.at[1 - b], isem)
            cpi[st].wait()
            adds[st] = [
                pltpu.async_copy(ones_v, acc.at[idx_v.at[b, j]], asem,
                                 add=True)
                for j in range(inner)
            ]
            if st >= 1:
                for a in adds[st - 1]:
                    a.wait()
        for a in adds[n_stage - 1]:
            a.wait()
        plsc.subcore_barrier()

        def dump(i, _):
            r0 = s * rpt + i * zb
            pltpu.sync_copy(acc.at[pl.ds(r0, zb), :], zbuf)
            pltpu.sync_copy(zbuf, out_hbm.at[c, pl.ds(r0, zb), :])
            return 0

        lax.fori_loop(0, n_zero, dump, 0)

    return k


def _dot(a, b):
    return jnp.dot(a, b, preferred_element_type=F32)


def _prep_body(x_ref, w_ref, b_ref, lw_ref, lb_ref, out0_ref, lin_ref):
    x = x_ref[...]
    out0_ref[...] = jnp.maximum(_dot(x, w_ref[...]) + b_ref[...], 0.0)
    lin_ref[...] = _dot(x, lw_ref[...]) + lb_ref[...]


def _prep(x, W, b, lW, lb):
    N, Fin = x.shape
    BN = 10000
    full = lambda r, c: pl.BlockSpec((r, c), lambda i: (0, 0))
    return pl.pallas_call(
        _prep_body,
        grid=(N // BN,),
        in_specs=[pl.BlockSpec((BN, Fin), lambda i: (i, 0)),
                  full(Fin, D), full(1, D), full(Fin, D), full(1, D)],
        out_specs=[pl.BlockSpec((BN, D), lambda i: (i, 0)),
                   pl.BlockSpec((BN, D), lambda i: (i, 0))],
        out_shape=[jax.ShapeDtypeStruct((N, D), F32),
                   jax.ShapeDtypeStruct((N, D), F32)],
    )(x, W, b, lW, lb)


def _msg_body(ea_ref, xj_ref, w1_ref, b1_ref, w2_ref, b2_ref, r_ref, s_ref,
              msg_ref):
    e1 = jnp.maximum(_dot(ea_ref[...], w1_ref[...]) + b1_ref[...], 0.0)
    e2 = jnp.maximum(_dot(e1, w2_ref[...]) + b2_ref[...], 0.0)
    xb = _dot(xj_ref[...], r_ref[...])
    msg_ref[...] = _dot(xb * e2, s_ref[...])


def _msg(ea, xj, W1, b1, W2, b2, Rm, Sm):
    E, EF = ea.shape
    BE = 5120
    full = lambda r, c: pl.BlockSpec((r, c), lambda i: (0, 0))
    return pl.pallas_call(
        _msg_body,
        grid=(E // BE,),
        in_specs=[pl.BlockSpec((BE, EF), lambda i: (i, 0)),
                  pl.BlockSpec((BE, D), lambda i: (i, 0)),
                  full(EF, D), full(1, D), full(D, D * D), full(1, D * D),
                  full(D, D * D), full(D * D, D)],
        out_specs=pl.BlockSpec((BE, D), lambda i: (i, 0)),
        out_shape=jax.ShapeDtypeStruct((E, D), F32),
    )(ea, xj, W1, b1, W2, b2, Rm, Sm)


def _gru_body(p_ref, cp_ref, h_ref, root_ref, bias_ref,
              wih_ref, whh_ref, lin_ref, out_ref, *, add_lin):
    h = h_ref[...]
    cnt = jnp.maximum(cp_ref[0] + cp_ref[1], 1.0)
    agg = (p_ref[0] + p_ref[1]) / cnt
    m = jnp.maximum(agg + _dot(h, root_ref[...]) + bias_ref[...], 0.0)
    gi = _dot(m, wih_ref[...])
    gh = _dot(h, whh_ref[...])
    r = jax.nn.sigmoid(gi[:, 0:D] + gh[:, 0:D])
    z = jax.nn.sigmoid(gi[:, D:2 * D] + gh[:, D:2 * D])
    n = jnp.tanh(gi[:, 2 * D:3 * D] + r * gh[:, 2 * D:3 * D])
    hn = (1.0 - z) * n + z * h
    if add_lin:
        hn = hn + lin_ref[...]
    out_ref[...] = hn


def _gru(parts, cparts, h, root, bias, WihT, WhhT, lin, add_lin):
    N = h.shape[0]
    BN = 5000
    full = lambda r, c: pl.BlockSpec((r, c), lambda i: (0, 0))
    body = functools.partial(_gru_body, add_lin=add_lin)
    return pl.pallas_call(
        body,
        grid=(N // BN,),
        in_specs=[pl.BlockSpec((2, BN, D), lambda i: (0, i, 0)),
                  pl.BlockSpec((2, BN, D), lambda i: (0, i, 0)),
                  pl.BlockSpec((BN, D), lambda i: (i, 0)),
                  full(D, D), full(1, D),
                  full(D, 3 * D), full(D, 3 * D),
                  pl.BlockSpec((BN, D), lambda i: (i, 0))],
        out_specs=pl.BlockSpec((BN, D), lambda i: (i, 0)),
        out_shape=jax.ShapeDtypeStruct((N, D), F32),
    )(parts, cparts, h, root, bias, WihT, WhhT, lin)


def _att_body(x_ref, lx0, lx1, lx2, lx3, lx4, lx5, lx6, lx7,
              mw_ref, mb_ref, lnw_ref, lnb_ref,
              wk_ref, bk_ref, wv_ref, bv_ref, wq_ref, bq_ref,
              wf_ref, bf_ref, hsel_ref, hselt_ref, xc_ref):
    lxs = (lx0, lx1, lx2, lx3, lx4, lx5, lx6, lx7)
    x = x_ref[...]
    xq = jnp.maximum(_dot(x, mw_ref[...]) + mb_ref[...], 0.0)
    q = (_dot(xq, wq_ref[...]) + bq_ref[...]) * 0.5
    hsel = hsel_ref[...]
    logits = []
    for j in range(8):
        kj = _dot(lxs[j][...], wk_ref[...]) + bk_ref[...]
        logits.append(_dot(q * kj, hsel))
    m = logits[0]
    for j in range(1, 8):
        m = jnp.maximum(m, logits[j])
    ps = [jnp.exp(l - m) for l in logits]
    den = ps[0]
    for j in range(1, 8):
        den = den + ps[j]
    hselt = hselt_ref[...]
    ctx = jnp.zeros_like(q)
    for j in range(8):
        vj = _dot(lxs[j][...], wv_ref[...]) + bv_ref[...]
        ctx = ctx + _dot(ps[j] / den, hselt) * vj
    low_ctx = _dot(ctx, wf_ref[...]) + bf_ref[...]
    left = jnp.maximum(_dot(x, lnw_ref[...]) + lnb_ref[...], 0.0)
    xc_ref[...] = jnp.concatenate([left, low_ctx], axis=1)


def _attention(x, lxs, mW, mb, lnW, lnb, att, Hsel, HselT):
    N = x.shape[0]
    BN = 2000
    full = lambda r, c: pl.BlockSpec((r, c), lambda i: (0, 0))
    blk = pl.BlockSpec((BN, D), lambda i: (i, 0))
    return pl.pallas_call(
        _att_body,
        grid=(N // BN,),
        in_specs=[blk] + [blk] * 8 +
                 [full(D, D), full(1, D), full(D, D), full(1, D),
                  full(D, D), full(1, D), full(D, D), full(1, D),
                  full(D, D), full(1, D), full(D, D), full(1, D),
                  full(D, 4), full(4, D)],
        out_specs=pl.BlockSpec((BN, 2 * D), lambda i: (i, 0)),
        out_shape=jax.ShapeDtypeStruct((N, 2 * D), F32),
    )(x, *lxs, mW, mb, lnW, lnb,
      att['Wk'], att['bk'][None], att['Wv'], att['bv'][None],
      att['Wq'], att['bq'][None], att['Wf'], att['bf'][None], Hsel, HselT)


def _s2s_body(x_ref,
              wii_ref, wif_ref, wig_ref, wio_ref,
              whi_ref, whf_ref, whg_ref, who_ref,
              bi_ref, bf_ref, bg_ref, bo_ref,
              m1w_ref, m1b_ref, m2w_ref, m2b_ref, out_ref, *, n_events):
    X = x_ref[...]
    N = X.shape[0]
    seg = N // n_events
    rI = lax.broadcasted_iota(jnp.int32, (n_events, N), 0)
    cI = lax.broadcasted_iota(jnp.int32, (n_events, N), 1)
    mb = rI == (cI // seg)
    q_star = jnp.zeros((n_events, 2 * D), F32)
    h = jnp.zeros((n_events, D), F32)
    c = jnp.zeros((n_events, D), F32)
    for _ in range(3):
        gi = _dot(q_star, wii_ref[...]) + _dot(h, whi_ref[...]) + bi_ref[...]
        gf = _dot(q_star, wif_ref[...]) + _dot(h, whf_ref[...]) + bf_ref[...]
        gg = _dot(q_star, wig_ref[...]) + _dot(h, whg_ref[...]) + bg_ref[...]
        go = _dot(q_star, wio_ref[...]) + _dot(h, who_ref[...]) + bo_ref[...]
        c = jax.nn.sigmoid(gf) * c + jax.nn.sigmoid(gi) * jnp.tanh(gg)
        h = jax.nn.sigmoid(go) * jnp.tanh(c)
        eT = lax.dot_general(h, X, (((1,), (1,)), ((), ())),
                             preferred_element_type=F32)
        e_m = jnp.where(mb, eT, -1e30)
        emax = jnp.max(e_m, axis=1, keepdims=True)
        p = jnp.where(mb, jnp.exp(eT - emax), 0.0)
        den = jnp.sum(p, axis=1, keepdims=True)
        r = _dot(p / den, X)
        q_star = jnp.concatenate([h, r], axis=1)
    o = jnp.maximum(_dot(q_star, m1w_ref[...]) + m1b_ref[...], 0.0)
    o2 = _dot(o, m2w_ref[...]) + m2b_ref[...]
    mx = jnp.max(o2, axis=1, keepdims=True)
    lse = jnp.log(jnp.sum(jnp.exp(o2 - mx), axis=1, keepdims=True)) + mx
    out_ref[...] = o2 - lse


def _set2set(h2, s2s, m1W, m1b, m2W, m2b, n_events):
    N = h2.shape[0]
    body = functools.partial(_s2s_body, n_events=n_events)
    full = lambda r, c: pl.BlockSpec((r, c), lambda i: (0, 0))
    WihT = s2s['Wih'].T
    WhhT = s2s['Whh'].T
    bsum = (s2s['bih'] + s2s['bhh'])[None]
    gates = [(WihT[:, g * D:(g + 1) * D], WhhT[:, g * D:(g + 1) * D],
              bsum[:, g * D:(g + 1) * D]) for g in range(4)]
    (wii, whi, bi), (wif, whf, bf), (wig, whg, bg), (wio, who, bo) = gates
    return pl.pallas_call(
        body,
        grid=(1,),
        in_specs=[full(N, D)] +
                 [full(2 * D, D)] * 4 + [full(D, D)] * 4 + [full(1, D)] * 4 +
                 [full(2 * D, D), full(1, D), full(D, 4), full(1, 4)],
        out_specs=full(n_events, 4),
        out_shape=jax.ShapeDtypeStruct((n_events, 4), F32),
    )(h2, wii, wif, wig, wio, whi, whf, whg, who, bi, bf, bg, bo,
      m1W, m1b[None], m2W, m2b[None])


def _fold_bn(W, b, g, bb):
    s = (g / jnp.sqrt(jnp.float32(1.0 + 1e-5))).astype(F32)
    return W * s[None, :], (b * s + bb)[None, :]


def _pad_edges(edge_index, edge_attr, E_pad, N):
    E = edge_index.shape[1]
    pad = E_pad - E
    src = jnp.concatenate([edge_index[0], jnp.zeros((pad,), jnp.int32)])
    dst = jnp.concatenate([edge_index[1], jnp.full((pad,), N, jnp.int32)])
    ea = jnp.concatenate([edge_attr, jnp.zeros((pad, edge_attr.shape[1]), F32)])
    return src.reshape(E_pad // IC, IC), dst.reshape(E_pad // IC, IC), ea


def _conv(x_nodes, src2d, dst2d, ea_pad, p, N, E_pad, N_acc):
    W1, b1 = _fold_bn(p['en1_W'], p['en1_b'], p['en_bn_g'], p['en_bn_b'])
    Wm, bm = _fold_bn(p['mlp_W'], p['mlp_b'], p['bn_g'], p['bn_b'])
    out0, lin = _prep(x_nodes, Wm, bm, p['lin_W'], p['lin_b'][None])
    cparts = _sc_count(E_pad, N, N_acc)(dst2d)
    di = jnp.arange(D * D, dtype=jnp.int32)
    Rm = (jnp.arange(D)[:, None] == (di[None, :] // D)).astype(F32)
    Sm = ((di[:, None] % D) == jnp.arange(D)[None, :]).astype(F32)
    gih = p['gru_Wih'].T
    ghh = p['gru_Whh'].T
    h = out0
    for step in range(3):
        xj = _sc_gather(E_pad, N)(h, src2d)
        msg = _msg(ea_pad, xj, W1, b1, p['en2_W'], p['en2_b'][None], Rm, Sm)
        parts = _sc_scatter_add(E_pad, N, N_acc)(msg, dst2d)
        h = _gru(parts, cparts, h, p['root'], p['bias'][None], gih, ghh,
                 lin, add_lin=(step == 2))
    return h


def kernel(x, edge_index, edge_attr, batch, low_x, low_edge_index,
           low_edge_attr, low_batch, params):
    p = params
    N_LOW = low_x.shape[0]
    N_HIGH = x.shape[0]
    n_events = 100
    EPL = 327680
    EPH = 163840
    NAL = 81920
    NAH = 11264

    lsrc, ldst, lea = _pad_edges(low_edge_index, low_edge_attr, EPL, N_LOW)
    hsrc, hdst, hea = _pad_edges(edge_index, edge_attr, EPH, N_HIGH)

    lx = _conv(low_x, lsrc, ldst, lea, p['c1'], N_LOW, EPL, NAL)
    lx3 = lx.reshape(N_HIGH, 8, D)
    lxs = [lx3[:, j, :] for j in range(8)]

    Hsel = ((jnp.arange(D)[:, None] // 4) == jnp.arange(4)[None, :]).astype(F32)
    xc = _attention(x, lxs, p['mlp_W'], p['mlp_b'][None],
                    p['ln_W'], p['ln_b'][None], p['att'], Hsel, Hsel.T)

    h2 = _conv(xc, hsrc, hdst, hea, p['c2'], N_HIGH, EPH, NAH)

    return _set2set(h2, p['s2s'], p['mlp1_W'], p['mlp1_b'],
                    p['mlp2_W'], p['mlp2_b'], n_events)


# speedup vs baseline: 5.2978x; 1.7506x over previous
import functools

import jax
import jax.numpy as jnp
from jax import lax
from jax.experimental import pallas as pl
from jax.experimental.pallas import tpu as pltpu
from jax.experimental.pallas import tpu_sc as plsc

F32 = jnp.float32
D = 16
IC = 128
STAGE = 1024
NW = 32


@functools.lru_cache(maxsize=None)
def _sc_gather(E_pad: int, N: int):
    per_tile = E_pad // NW
    n_stage = per_tile // STAGE
    inner = STAGE // IC
    mesh = plsc.VectorSubcoreMesh(core_axis_name="c", subcore_axis_name="s")

    @functools.partial(
        pl.kernel,
        out_type=jax.ShapeDtypeStruct((E_pad, D), F32),
        mesh=mesh,
        compiler_params=pltpu.CompilerParams(use_tc_tiling_on_sc=False),
        scratch_types=[
            pltpu.VMEM((2, inner, IC), jnp.int32),
            pltpu.VMEM((2, STAGE, D), F32),
            pltpu.SemaphoreType.DMA,
            pltpu.SemaphoreType.DMA,
            pltpu.SemaphoreType.DMA,
        ],
    )
    def k(table_hbm, idx_hbm, out_hbm, idx_v, rows_v, isem, gsem, osem):
        wid = lax.axis_index("c") * 16 + lax.axis_index("s")
        base_irow = wid * (per_tile // IC)

        cpi = {0: pltpu.async_copy(idx_hbm.at[pl.ds(base_irow, inner), :],
                                   idx_v.at[0], isem)}
        cpo = {}
        for st in range(n_stage):
            b = st % 2
            if st + 1 < n_stage:
                cpi[st + 1] = pltpu.async_copy(
                    idx_hbm.at[pl.ds(base_irow + (st + 1) * inner, inner), :],
                    idx_v.at[1 - b], isem)
            cpi[st].wait()
            if st >= 2:
                cpo[st - 2].wait()
            gs = [
                pltpu.async_copy(
                    table_hbm.at[idx_v.at[b, j]],
                    rows_v.at[b, pl.ds(j * IC, IC), :],
                    gsem,
                )
                for j in range(inner)
            ]
            for g in gs:
                g.wait()
            cpo[st] = pltpu.async_copy(
                rows_v.at[b],
                out_hbm.at[pl.ds(wid * per_tile + st * STAGE, STAGE), :],
                osem)
        for st in range(max(0, n_stage - 2), n_stage):
            cpo[st].wait()

    return k


@functools.lru_cache(maxsize=None)
def _sc_scatter_add(E_pad: int, N: int, N_acc: int):
    per_tile = E_pad // NW
    n_stage = per_tile // STAGE
    inner = STAGE // IC
    rpt = N_acc // 16
    zb = 256 if rpt % 256 == 0 else rpt // 4
    n_zero = rpt // zb
    dpt = N_acc // 16
    db = min(dpt, STAGE)
    n_dump = dpt // db
    mesh = plsc.VectorSubcoreMesh(core_axis_name="c", subcore_axis_name="s")

    @functools.partial(
        pl.kernel,
        out_type=jax.ShapeDtypeStruct((2, N_acc, D), F32),
        mesh=mesh,
        compiler_params=pltpu.CompilerParams(use_tc_tiling_on_sc=False),
        scratch_types=[
            pltpu.VMEM((2, inner, IC), jnp.int32),
            pltpu.VMEM((2, STAGE, D), F32),
            pltpu.VMEM((zb, D), F32),
            pltpu.VMEM_SHARED((N_acc, D), F32),
            pltpu.SemaphoreType.DMA,
            pltpu.SemaphoreType.DMA,
            pltpu.SemaphoreType.DMA,
        ],
    )
    def k(msg_hbm, idx_hbm, out_hbm, idx_v, rows_v, zbuf, acc, isem, asem, osem):
        c = lax.axis_index("c")
        s = lax.axis_index("s")
        wid = c * 16 + s

        def zfill(i, _):
            zbuf[i] = jnp.zeros((D,), F32)
            return 0
        lax.fori_loop(0, zb, zfill, 0)
        zcs = [pltpu.async_copy(zbuf, acc.at[pl.ds(s * rpt + i * zb, zb), :],
                                osem)
               for i in range(n_zero)]
        for z in zcs:
            z.wait()
        plsc.subcore_barrier()

        base_irow = wid * (per_tile // IC)

        def load(st, b):
            ci = pltpu.async_copy(
                idx_hbm.at[pl.ds(base_irow + st * inner, inner), :],
                idx_v.at[b], isem)
            cm = pltpu.async_copy(
                msg_hbm.at[pl.ds(wid * per_tile + st * STAGE, STAGE), :],
                rows_v.at[b], osem)
            return ci, cm

        pend = {0: load(0, 0)}
        adds = {}
        for st in range(n_stage):
            b = st % 2
            ci, cm = pend[st]
            ci.wait()
            cm.wait()
            adds[st] = [
                pltpu.async_copy(
                    rows_v.at[b, pl.ds(j * IC, IC), :],
                    acc.at[idx_v.at[b, j]],
                    asem, add=True)
                for j in range(inner)
            ]
            if st >= 1:
                for a in adds[st - 1]:
                    a.wait()
            if st + 1 < n_stage:
                pend[st + 1] = load(st + 1, 1 - b)
        for a in adds[n_stage - 1]:
            a.wait()
        plsc.subcore_barrier()

        cpo = {}
        for i in range(n_dump):
            b = i % 2
            r0 = s * dpt + i * db
            if i >= 2:
                cpo[i - 2].wait()
            pltpu.sync_copy(acc.at[pl.ds(r0, db), :], rows_v.at[b, pl.ds(0, db), :])
            cpo[i] = pltpu.async_copy(rows_v.at[b, pl.ds(0, db), :],
                                      out_hbm.at[c, pl.ds(r0, db), :], osem)
        for i in range(max(0, n_dump - 2), n_dump):
            cpo[i].wait()

    return k


@functools.lru_cache(maxsize=None)
def _sc_count(E_pad: int, N: int, N_acc: int):
    per_tile = E_pad // NW
    n_stage = per_tile // STAGE
    inner = STAGE // IC
    rpt = N_acc // 16
    zb = 256 if rpt % 256 == 0 else rpt // 4
    n_zero = rpt // zb
    mesh = plsc.VectorSubcoreMesh(core_axis_name="c", subcore_axis_name="s")

    @functools.partial(
        pl.kernel,
        out_type=jax.ShapeDtypeStruct((2, N_acc, D), F32),
        mesh=mesh,
        compiler_params=pltpu.CompilerParams(use_tc_tiling_on_sc=False),
        scratch_types=[
            pltpu.VMEM((2, inner, IC), jnp.int32),
            pltpu.VMEM((IC, D), F32),
            pltpu.VMEM((zb, D), F32),
            pltpu.VMEM_SHARED((N_acc, D), F32),
            pltpu.SemaphoreType.DMA,
            pltpu.SemaphoreType.DMA,
            pltpu.SemaphoreType.DMA,
        ],
    )
    def k(idx_hbm, out_hbm, idx_v, ones_v, zbuf, acc, isem, asem, osem):
        c = lax.axis_index("c")
        s = lax.axis_index("s")
        wid = c * 16 + s

        def ofill(i, _):
            ones_v[i] = jnp.ones((D,), F32)
            return 0
        lax.fori_loop(0, IC, ofill, 0)

        def zfill(i, _):
            zbuf[i] = jnp.zeros((D,), F32)
            return 0
        lax.fori_loop(0, zb, zfill, 0)

        zcs = [pltpu.async_copy(zbuf, acc.at[pl.ds(s * rpt + i * zb, zb), :],
                                osem)
               for i in range(n_zero)]
        for z in zcs:
            z.wait()
        plsc.subcore_barrier()

        base_irow = wid * (per_tile // IC)
        cpi = {0: pltpu.async_copy(idx_hbm.at[pl.ds(base_irow, inner), :],
                                   idx_v.at[0], isem)}
        adds = {}
        for st in range(n_stage):
            b = st % 2
            if st + 1 < n_stage:
                cpi[st + 1] = pltpu.async_copy(
                    idx_hbm.at[pl.ds(base_irow + (st + 1) * inner, inner), :],
                    idx_v.at[1 - b], isem)
            cpi[st].wait()
            adds[st] = [
                pltpu.async_copy(ones_v, acc.at[idx_v.at[b, j]], asem,
                                 add=True)
                for j in range(inner)
            ]
            if st >= 1:
                for a in adds[st - 1]:
                    a.wait()
        for a in adds[n_stage - 1]:
            a.wait()
        plsc.subcore_barrier()

        def dump(i, _):
            r0 = s * rpt + i * zb
            pltpu.sync_copy(acc.at[pl.ds(r0, zb), :], zbuf)
            pltpu.sync_copy(zbuf, out_hbm.at[c, pl.ds(r0, zb), :])
            return 0

        lax.fori_loop(0, n_zero, dump, 0)

    return k


def _dot(a, b):
    return jnp.dot(a, b, preferred_element_type=F32)


def _k8(W):
    return jnp.kron(jnp.eye(8, dtype=F32), W.astype(F32))


def _t8(b):
    return jnp.tile(b.reshape(1, -1).astype(F32), (1, 8))


def _prep_body(x_ref, w_ref, b_ref, lw_ref, lb_ref, out0_ref, lin_ref):
    x = x_ref[...]
    out0_ref[...] = jnp.maximum(_dot(x, w_ref[...]) + b_ref[...], 0.0)
    lin_ref[...] = _dot(x, lw_ref[...]) + lb_ref[...]


def _prep(x_p, Wp, bp, lWp, lbp):
    N8, F8 = x_p.shape
    BN = N8 if N8 % 2000 else 2000
    full = lambda r, c: pl.BlockSpec((r, c), lambda i: (0, 0))
    return pl.pallas_call(
        _prep_body,
        grid=(N8 // BN,),
        in_specs=[pl.BlockSpec((BN, F8), lambda i: (i, 0)),
                  full(F8, 128), full(1, 128), full(F8, 128), full(1, 128)],
        out_specs=[pl.BlockSpec((BN, 128), lambda i: (i, 0)),
                   pl.BlockSpec((BN, 128), lambda i: (i, 0))],
        out_shape=[jax.ShapeDtypeStruct((N8, 128), F32),
                   jax.ShapeDtypeStruct((N8, 128), F32)],
    )(x_p, Wp, bp, lWp, lbp)


def _msg_body(ea_ref, xj_ref, w1_ref, b1_ref, w2_ref, b2_ref, r_ref, s_ref,
              msg_ref):
    e1 = jnp.maximum(_dot(ea_ref[...], w1_ref[...]) + b1_ref[...], 0.0)
    e2 = jnp.maximum(_dot(e1, w2_ref[...]) + b2_ref[...], 0.0)
    xb = _dot(xj_ref[...], r_ref[...])
    msg_ref[...] = _dot(xb * e2, s_ref[...])


def _msg(ea_p, xj_p, W1p, b1p, W2p, b2p, Rp, Sp):
    E8 = ea_p.shape[0]
    BE = 640
    full = lambda r, c: pl.BlockSpec((r, c), lambda i: (0, 0))
    return pl.pallas_call(
        _msg_body,
        grid=(E8 // BE,),
        in_specs=[pl.BlockSpec((BE, 128), lambda i: (i, 0)),
                  pl.BlockSpec((BE, 128), lambda i: (i, 0)),
                  full(128, 128), full(1, 128),
                  full(128, 8 * D * D), full(1, 8 * D * D),
                  full(128, 8 * D * D), full(8 * D * D, 128)],
        out_specs=pl.BlockSpec((BE, 128), lambda i: (i, 0)),
        out_shape=jax.ShapeDtypeStruct((E8, 128), F32),
    )(ea_p, xj_p, W1p, b1p, W2p, b2p, Rp, Sp)


def _gru_body(p_ref, cp_ref, h_ref, root_ref, bias_ref,
              wir_ref, wiz_ref, win_ref, whr_ref, whz_ref, whn_ref,
              lin_ref, out_ref, *, add_lin, rows):
    h = h_ref[...]
    p0, p1 = p_ref[0], p_ref[1]
    c0, c1 = cp_ref[0], cp_ref[1]
    if rows is not None:
        p0, p1 = p0[:rows], p1[:rows]
        c0, c1 = c0[:rows], c1[:rows]
    cnt = jnp.maximum(c0 + c1, 1.0)
    agg = (p0 + p1) / cnt
    m = jnp.maximum(agg + _dot(h, root_ref[...]) + bias_ref[...], 0.0)
    r = jax.nn.sigmoid(_dot(m, wir_ref[...]) + _dot(h, whr_ref[...]))
    z = jax.nn.sigmoid(_dot(m, wiz_ref[...]) + _dot(h, whz_ref[...]))
    n = jnp.tanh(_dot(m, win_ref[...]) + r * _dot(h, whn_ref[...]))
    hn = (1.0 - z) * n + z * h
    if add_lin:
        hn = hn + lin_ref[...]
    out_ref[...] = hn


def _gru(parts_p, cparts_p, h_p, rootp, biasp, gih, ghh, lin_p, add_lin):
    N8 = h_p.shape[0]
    NA8 = parts_p.shape[1]
    full = lambda r, c: pl.BlockSpec((r, c), lambda i: (0, 0))
    if N8 % 2000 == 0:
        BN, rows = 2000, None
        pspec = pl.BlockSpec((2, BN, 128), lambda i: (0, i, 0))
    else:
        BN, rows = N8, N8
        pspec = pl.BlockSpec((2, NA8, 128), lambda i: (0, 0, 0))
    body = functools.partial(_gru_body, add_lin=add_lin, rows=rows)
    return pl.pallas_call(
        body,
        grid=(N8 // BN,),
        in_specs=[pspec,
                  pspec,
                  pl.BlockSpec((BN, 128), lambda i: (i, 0)),
                  full(128, 128), full(1, 128),
                  full(128, 128), full(128, 128), full(128, 128),
                  full(128, 128), full(128, 128), full(128, 128),
                  pl.BlockSpec((BN, 128), lambda i: (i, 0))],
        out_specs=pl.BlockSpec((BN, 128), lambda i: (i, 0)),
        out_shape=jax.ShapeDtypeStruct((N8, 128), F32),
    )(parts_p, cparts_p, h_p, rootp, biasp, *gih, *ghh, lin_p)


def _att_body(x_ref, lx_ref, mw_ref, mb_ref, lnw_ref, lnb_ref,
              wk_refs, bk_ref, wv_refs, bv_ref, wq_ref, bq_ref,
              wf_ref, bf_ref, hsel_ref, hselt_ref, p1_ref, p2_ref, xc_ref):
    x = x_ref[...]
    lx = lx_ref[...]
    xq = jnp.maximum(_dot(x, mw_ref[...]) + mb_ref[...], 0.0)
    q = (_dot(xq, wq_ref[...]) + bq_ref[...]) * 0.5
    hsel = hsel_ref[...]
    logits = []
    for j in range(8):
        kj = _dot(lx, wk_refs[j][...]) + bk_ref[...]
        logits.append(_dot(q * kj, hsel))
    m = logits[0]
    for j in range(1, 8):
        m = jnp.maximum(m, logits[j])
    ps = [jnp.exp(l - m) for l in logits]
    den = ps[0]
    for j in range(1, 8):
        den = den + ps[j]
    hselt = hselt_ref[...]
    ctx = jnp.zeros_like(q)
    for j in range(8):
        vj = _dot(lx, wv_refs[j][...]) + bv_ref[...]
        ctx = ctx + _dot(ps[j] / den, hselt) * vj
    low_ctx = _dot(ctx, wf_ref[...]) + bf_ref[...]
    left = jnp.maximum(_dot(x, lnw_ref[...]) + lnb_ref[...], 0.0)
    xc_ref[...] = _dot(left, p1_ref[...]) + _dot(low_ctx, p2_ref[...])


def _att_entry(x_ref, lx_ref, mw, mb, lnw, lnb,
               wk0, wk1, wk2, wk3, wk4, wk5, wk6, wk7, bk,
               wv0, wv1, wv2, wv3, wv4, wv5, wv6, wv7, bv,
               wq, bq, wf, bf, hsel, hselt, p1, p2, xc_ref):
    _att_body(x_ref, lx_ref, mw, mb, lnw, lnb,
              (wk0, wk1, wk2, wk3, wk4, wk5, wk6, wk7), bk,
              (wv0, wv1, wv2, wv3, wv4, wv5, wv6, wv7), bv,
              wq, bq, wf, bf, hsel, hselt, p1, p2, xc_ref)


def _attention(x_p, lx1024, mWp, mbp, lnWp, lnbp, att, Hselp, HselTp, P1, P2):
    N8 = x_p.shape[0]
    full = lambda r, c: pl.BlockSpec((r, c), lambda i: (0, 0))
    Wk, Wv = att['Wk'], att['Wv']
    WKs = []
    WVs = []
    for j in range(8):
        Ej = jnp.zeros((128, D), F32).at[16 * j:16 * j + D, :].set(
            jnp.eye(D, dtype=F32))
        WKs.append(_k8(Ej @ Wk))
        WVs.append(_k8(Ej @ Wv))
    return pl.pallas_call(
        _att_entry,
        grid=(1,),
        in_specs=[full(N8, 128), full(N8, 1024),
                  full(128, 128), full(1, 128), full(128, 128), full(1, 128)]
                 + [full(1024, 128)] * 8 + [full(1, 128)]
                 + [full(1024, 128)] * 8 + [full(1, 128)]
                 + [full(128, 128), full(1, 128), full(128, 128), full(1, 128),
                    full(128, 32), full(32, 128), full(128, 256),
                    full(128, 256)],
        out_specs=full(N8, 256),
        out_shape=jax.ShapeDtypeStruct((N8, 256), F32),
    )(x_p, lx1024, mWp, mbp, lnWp, lnbp,
      *WKs, _t8(att['bk']), *WVs, _t8(att['bv']),
      _k8(att['Wq']), _t8(att['bq']), _k8(att['Wf']), _t8(att['bf']),
      Hselp, HselTp, P1, P2)


def _s2s_body(x_ref,
              wii_ref, wif_ref, wig_ref, wio_ref,
              whi_ref, whf_ref, whg_ref, who_ref,
              bi_ref, bf_ref, bg_ref, bo_ref,
              m1w_ref, m1b_ref, m2w_ref, m2b_ref, out_ref, *, n_events):
    X = x_ref[...]
    N = X.shape[0]
    seg = N // n_events
    rI = lax.broadcasted_iota(jnp.int32, (n_events, N), 0)
    cI = lax.broadcasted_iota(jnp.int32, (n_events, N), 1)
    mb = rI == (cI // seg)
    q_star = jnp.zeros((n_events, 2 * D), F32)
    h = jnp.zeros((n_events, D), F32)
    c = jnp.zeros((n_events, D), F32)
    for _ in range(3):
        gi = _dot(q_star, wii_ref[...]) + _dot(h, whi_ref[...]) + bi_ref[...]
        gf = _dot(q_star, wif_ref[...]) + _dot(h, whf_ref[...]) + bf_ref[...]
        gg = _dot(q_star, wig_ref[...]) + _dot(h, whg_ref[...]) + bg_ref[...]
        go = _dot(q_star, wio_ref[...]) + _dot(h, who_ref[...]) + bo_ref[...]
        c = jax.nn.sigmoid(gf) * c + jax.nn.sigmoid(gi) * jnp.tanh(gg)
        h = jax.nn.sigmoid(go) * jnp.tanh(c)
        eT = lax.dot_general(h, X, (((1,), (1,)), ((), ())),
                             preferred_element_type=F32)
        e_m = jnp.where(mb, eT, -1e30)
        emax = jnp.max(e_m, axis=1, keepdims=True)
        p = jnp.where(mb, jnp.exp(eT - emax), 0.0)
        den = jnp.sum(p, axis=1, keepdims=True)
        r = _dot(p / den, X)
        q_star = jnp.concatenate([h, r], axis=1)
    o = jnp.maximum(_dot(q_star, m1w_ref[...]) + m1b_ref[...], 0.0)
    o2 = _dot(o, m2w_ref[...]) + m2b_ref[...]
    mx = jnp.max(o2, axis=1, keepdims=True)
    lse = jnp.log(jnp.sum(jnp.exp(o2 - mx), axis=1, keepdims=True)) + mx
    out_ref[...] = o2 - lse


def _set2set(h2, s2s, m1W, m1b, m2W, m2b, n_events):
    N = h2.shape[0]
    body = functools.partial(_s2s_body, n_events=n_events)
    full = lambda r, c: pl.BlockSpec((r, c), lambda i: (0, 0))
    WihT = s2s['Wih'].T
    WhhT = s2s['Whh'].T
    bsum = (s2s['bih'] + s2s['bhh'])[None]
    gates = [(WihT[:, g * D:(g + 1) * D], WhhT[:, g * D:(g + 1) * D],
              bsum[:, g * D:(g + 1) * D]) for g in range(4)]
    (wii, whi, bi), (wif, whf, bf), (wig, whg, bg), (wio, who, bo) = gates
    return pl.pallas_call(
        body,
        grid=(1,),
        in_specs=[full(N, D)] +
                 [full(2 * D, D)] * 4 + [full(D, D)] * 4 + [full(1, D)] * 4 +
                 [full(2 * D, D), full(1, D), full(D, 4), full(1, 4)],
        out_specs=full(n_events, 4),
        out_shape=jax.ShapeDtypeStruct((n_events, 4), F32),
    )(h2, wii, wif, wig, wio, whi, whf, whg, who, bi, bf, bg, bo,
      m1W, m1b[None], m2W, m2b[None])


def _fold_bn(W, b, g, bb):
    s = (g / jnp.sqrt(jnp.float32(1.0 + 1e-5))).astype(F32)
    return W * s[None, :], b * s + bb


def _pad_edges(edge_index, edge_attr, E_pad, N):
    E = edge_index.shape[1]
    pad = E_pad - E
    src = jnp.concatenate([edge_index[0], jnp.zeros((pad,), jnp.int32)])
    dst = jnp.concatenate([edge_index[1], jnp.full((pad,), N, jnp.int32)])
    ea16 = jnp.pad(edge_attr, ((0, pad), (0, D - edge_attr.shape[1])))
    return (src.reshape(E_pad // IC, IC), dst.reshape(E_pad // IC, IC),
            ea16.reshape(E_pad // 8, 128))


def _conv(x_p, src2d, dst2d, ea_p, p, N, E_pad, N_acc):
    W1, b1 = _fold_bn(p['en1_W'], p['en1_b'], p['en_bn_g'], p['en_bn_b'])
    W1 = jnp.zeros((D, D), F32).at[:W1.shape[0], :].set(W1)
    Wm, bm = _fold_bn(p['mlp_W'], p['mlp_b'], p['bn_g'], p['bn_b'])
    out0_p, lin_p = _prep(x_p, _k8(Wm), _t8(bm), _k8(p['lin_W']),
                          _t8(p['lin_b']))
    cparts = _sc_count(E_pad, N, N_acc)(dst2d)
    cparts_p = cparts.reshape(2, N_acc // 8, 128)
    di = jnp.arange(D * D, dtype=jnp.int32)
    Rm = (jnp.arange(D)[:, None] == (di[None, :] // D)).astype(F32)
    Sm = ((di[:, None] % D) == jnp.arange(D)[None, :]).astype(F32)
    WihT = p['gru_Wih'].T
    WhhT = p['gru_Whh'].T
    gih = tuple(_k8(WihT[:, g * D:(g + 1) * D]) for g in range(3))
    ghh = tuple(_k8(WhhT[:, g * D:(g + 1) * D]) for g in range(3))
    rootp = _k8(p['root'])
    biasp = _t8(p['bias'])
    W1p = _k8(W1)
    b1p = _t8(b1)
    W2p = _k8(p['en2_W'])
    b2p = _t8(p['en2_b'])
    Rp = _k8(Rm)
    Sp = _k8(Sm)
    h_p = out0_p
    for step in range(3):
        xj = _sc_gather(E_pad, N)(h_p.reshape(N, D), src2d)
        msg_p = _msg(ea_p, xj.reshape(E_pad // 8, 128),
                     W1p, b1p, W2p, b2p, Rp, Sp)
        parts = _sc_scatter_add(E_pad, N, N_acc)(msg_p.reshape(E_pad, D),
                                                 dst2d)
        h_p = _gru(parts.reshape(2, N_acc // 8, 128), cparts_p, h_p,
                   rootp, biasp, gih, ghh, lin_p, add_lin=(step == 2))
    return h_p


def kernel(x, edge_index, edge_attr, batch, low_x, low_edge_index,
           low_edge_attr, low_batch, params):
    p = params
    N_LOW = low_x.shape[0]
    N_HIGH = x.shape[0]
    n_events = 100
    EPL = 327680
    EPH = 163840
    NAL = 81920
    NAH = 11264

    lsrc, ldst, lea = _pad_edges(low_edge_index, low_edge_attr, EPL, N_LOW)
    hsrc, hdst, hea = _pad_edges(edge_index, edge_attr, EPH, N_HIGH)

    lx_p = _conv(low_x.reshape(N_LOW // 8, 128), lsrc, ldst, lea,
                 p['c1'], N_LOW, EPL, NAL)

    Hsel = ((jnp.arange(D)[:, None] // 4) == jnp.arange(4)[None, :]).astype(F32)
    A1 = jnp.concatenate([jnp.eye(D, dtype=F32),
                          jnp.zeros((D, D), F32)], axis=1)
    A2 = jnp.concatenate([jnp.zeros((D, D), F32),
                          jnp.eye(D, dtype=F32)], axis=1)
    xc_p = _attention(x.reshape(N_HIGH // 8, 128),
                      lx_p.reshape(N_HIGH // 8, 1024),
                      _k8(p['mlp_W']), _t8(p['mlp_b']),
                      _k8(p['ln_W']), _t8(p['ln_b']), p['att'],
                      _k8(Hsel), _k8(Hsel.T), _k8(A1), _k8(A2))

    h2_p = _conv(xc_p, hsrc, hdst, hea, p['c2'], N_HIGH, EPH, NAH)

    return _set2set(h2_p.reshape(N_HIGH, D), p['s2s'], p['mlp1_W'],
                    p['mlp1_b'], p['mlp2_W'], p['mlp2_b'], n_events)
